# Initial kernel scaffold; baseline (speedup 1.0000x reference)
import jax
import jax.numpy as jnp
from jax.experimental import pallas as pl


def kernel(t, gamma):
    raise NotImplementedError("write your pallas kernel here")


# speedup vs baseline: 4.5077x; 4.5077x over previous
import functools

import jax
import jax.numpy as jnp
from jax import lax
from jax.experimental import pallas as pl
from jax.experimental.pallas import tpu as pltpu
from jax.experimental.pallas import tpu_sc as plsc

_TIMESTEPS_SCALE = 1000.0
_RNE_MAGIC = 8388608.0
_LANES = 16

_B = 16384
_GAMMA_PAD = 1024


def _body(nw, b_per_w, t_hbm, gamma_hbm, out_hbm, gamma_v, t_v, out_v):
    wid = lax.axis_index("s") * 2 + lax.axis_index("c")
    base = wid * b_per_w
    pltpu.sync_copy(gamma_hbm, gamma_v)
    pltpu.sync_copy(t_hbm.at[pl.ds(base, b_per_w)], t_v)
    for i in range(b_per_w // _LANES):
        x = t_v[pl.ds(i * _LANES, _LANES)]
        y = (x * _TIMESTEPS_SCALE + _RNE_MAGIC) - _RNE_MAGIC
        idx = y.astype(jnp.int32)
        out_v[pl.ds(i * _LANES, _LANES)] = plsc.load_gather(gamma_v, [idx])
    pltpu.sync_copy(out_v, out_hbm.at[pl.ds(base, b_per_w)])


def kernel(t, gamma):
    gamma_p = jnp.pad(gamma, (0, _GAMMA_PAD - gamma.shape[0]))
    info = plsc.get_sparse_core_info()
    nw = info.num_cores * info.num_subcores
    b_per_w = _B // nw
    mesh = plsc.VectorSubcoreMesh(core_axis_name="c", subcore_axis_name="s")
    k = functools.partial(
        pl.kernel,
        mesh=mesh,
        out_type=jax.ShapeDtypeStruct((_B,), jnp.float32),
        scratch_types=[
            pltpu.VMEM((_GAMMA_PAD,), jnp.float32),
            pltpu.VMEM((b_per_w,), jnp.float32),
            pltpu.VMEM((b_per_w,), jnp.float32),
        ],
        compiler_params=pltpu.CompilerParams(needs_layout_passes=False),
    )(functools.partial(_body, nw, b_per_w))
    return k(t, gamma_p)


# speedup vs baseline: 4.6095x; 1.0226x over previous
import functools

import jax
import jax.numpy as jnp
from jax import lax
from jax.experimental import pallas as pl
from jax.experimental.pallas import tpu as pltpu
from jax.experimental.pallas import tpu_sc as plsc

_TIMESTEPS_SCALE = 1000.0
_RNE_MAGIC = 8388608.0
_LANES = 16

_B = 16384


def _body(b_per_w, t_hbm, gamma_hbm, out_hbm, gamma_v, t_v, out_v, sem_g, sem_t):
    wid = lax.axis_index("s") * 2 + lax.axis_index("c")
    base = wid * b_per_w
    cp_g = pltpu.async_copy(gamma_hbm, gamma_v, sem_g)
    cp_t = pltpu.async_copy(t_hbm.at[pl.ds(base, b_per_w)], t_v, sem_t)
    cp_g.wait()
    cp_t.wait()
    for i in range(b_per_w // _LANES):
        x = t_v[pl.ds(i * _LANES, _LANES)]
        y = (x * _TIMESTEPS_SCALE + _RNE_MAGIC) - _RNE_MAGIC
        idx = y.astype(jnp.int32)
        out_v[pl.ds(i * _LANES, _LANES)] = plsc.load_gather(gamma_v, [idx])
    pltpu.sync_copy(out_v, out_hbm.at[pl.ds(base, b_per_w)])


def kernel(t, gamma):
    info = plsc.get_sparse_core_info()
    nw = info.num_cores * info.num_subcores
    b_per_w = _B // nw
    mesh = plsc.VectorSubcoreMesh(core_axis_name="c", subcore_axis_name="s")
    k = functools.partial(
        pl.kernel,
        mesh=mesh,
        out_type=jax.ShapeDtypeStruct((_B,), jnp.float32),
        scratch_types=[
            pltpu.VMEM(gamma.shape, jnp.float32),
            pltpu.VMEM((b_per_w,), jnp.float32),
            pltpu.VMEM((b_per_w,), jnp.float32),
            pltpu.SemaphoreType.DMA,
            pltpu.SemaphoreType.DMA,
        ],
        compiler_params=pltpu.CompilerParams(needs_layout_passes=False),
    )(functools.partial(_body, b_per_w))
    return k(t, gamma)


# speedup vs baseline: 4.9906x; 1.0827x over previous
import functools

import jax
import jax.numpy as jnp
from jax import lax
from jax.experimental import pallas as pl
from jax.experimental.pallas import tpu as pltpu
from jax.experimental.pallas import tpu_sc as plsc

_TIMESTEPS_SCALE = 1000.0
_RNE_MAGIC = 8388608.0
_LANES = 16

_B = 16384


def _body(b_per_w, t_hbm, gamma_hbm, out_hbm, gamma_v, t_v, out_v, sem_g, sem_t):
    wid = lax.axis_index("s")
    base = wid * b_per_w
    cp_g = pltpu.async_copy(gamma_hbm, gamma_v, sem_g)
    cp_t = pltpu.async_copy(t_hbm.at[pl.ds(base, b_per_w)], t_v, sem_t)
    cp_g.wait()
    cp_t.wait()
    for i in range(b_per_w // _LANES):
        x = t_v[pl.ds(i * _LANES, _LANES)]
        y = (x * _TIMESTEPS_SCALE + _RNE_MAGIC) - _RNE_MAGIC
        idx = y.astype(jnp.int32)
        out_v[pl.ds(i * _LANES, _LANES)] = plsc.load_gather(gamma_v, [idx])
    pltpu.sync_copy(out_v, out_hbm.at[pl.ds(base, b_per_w)])


def kernel(t, gamma):
    info = plsc.get_sparse_core_info()
    nw = info.num_subcores
    b_per_w = _B // nw
    mesh = plsc.VectorSubcoreMesh(
        core_axis_name="c", subcore_axis_name="s", num_cores=1
    )
    k = functools.partial(
        pl.kernel,
        mesh=mesh,
        out_type=jax.ShapeDtypeStruct((_B,), jnp.float32),
        scratch_types=[
            pltpu.VMEM(gamma.shape, jnp.float32),
            pltpu.VMEM((b_per_w,), jnp.float32),
            pltpu.VMEM((b_per_w,), jnp.float32),
            pltpu.SemaphoreType.DMA,
            pltpu.SemaphoreType.DMA,
        ],
        compiler_params=pltpu.CompilerParams(needs_layout_passes=False),
    )(functools.partial(_body, b_per_w))
    return k(t, gamma)
